# baseline (speedup 1.0000x reference)
import functools

import jax
import jax.numpy as jnp
from jax import lax
from jax.experimental import pallas as pl
from jax.experimental.pallas import tpu as pltpu
from jax.experimental.pallas import tpu_sc as plsc

T, D, E, F, TOP_K = 2048, 1024, 8, 2048, 2
TM = 512
NT = T * TOP_K // TM + E
NTOT = NT * TM
NSC = 16
TT = T // NSC
CH = 64
L = 16


def _router_body(x_ref, wr_ref, code_ref, w1_ref):
    x = x_ref[...]
    logits = jnp.dot(x, wr_ref[...], preferred_element_type=jnp.float32)
    lane = lax.broadcasted_iota(jnp.int32, logits.shape, 1)
    logits = jnp.where(lane < E, logits, -jnp.inf)
    m1 = jnp.max(logits, axis=1, keepdims=True)
    i1 = jnp.min(jnp.where(logits == m1, lane, E), axis=1, keepdims=True)
    logits2 = jnp.where(lane == i1, -jnp.inf, logits)
    m2 = jnp.max(logits2, axis=1, keepdims=True)
    i2 = jnp.min(jnp.where(logits2 == m2, lane, E), axis=1, keepdims=True)
    w1 = 1.0 / (1.0 + jnp.exp(m2 - m1))
    code_ref[...] = i1 * E + i2
    w1_ref[...] = w1


def _router(x_TD, W_router_pad):
    return pl.pallas_call(
        _router_body,
        grid=(T // TM,),
        in_specs=[
            pl.BlockSpec((TM, D), lambda i: (i, 0)),
            pl.BlockSpec((D, 128), lambda i: (0, 0)),
        ],
        out_specs=[
            pl.BlockSpec((TM, 1), lambda i: (i, 0)),
            pl.BlockSpec((TM, 1), lambda i: (i, 0)),
        ],
        out_shape=[
            jax.ShapeDtypeStruct((T, 1), jnp.int32),
            jax.ShapeDtypeStruct((T, 1), jnp.float32),
        ],
    )(x_TD, W_router_pad)


def _dispatch_body(code_hbm, x_hbm,
                   xs_hbm, s0_hbm, s1_hbm, te_hbm, tv_hbm,
                   code_v, hist_v, allhist_v, s0_v, s1_v, rows_v,
                   te_v, tv_v, shared_hist, sem0, sem1):
    wid = lax.axis_index("s")
    base_tok = wid * TT
    iota = lax.iota(jnp.int32, L)

    pltpu.sync_copy(code_hbm.at[pl.ds(base_tok, TT)], code_v)

    accs = [jnp.zeros((L,), jnp.int32) for _ in range(E)]
    for j in range(TT // L):
        c16 = code_v[pl.ds(j * L, L)]
        e0 = c16 // E
        e1 = c16 - e0 * E
        for e in range(E):
            accs[e] = accs[e] + (e0 == e).astype(jnp.int32) + (e1 == e).astype(jnp.int32)
    hv = jnp.zeros((L,), jnp.int32)
    for e in range(E):
        hv = jnp.where(iota == e, jnp.sum(accs[e]), hv)
    hist_v[...] = hv

    pltpu.sync_copy(hist_v, shared_hist.at[pl.ds(wid * L, L)])
    plsc.subcore_barrier()
    pltpu.sync_copy(shared_hist, allhist_v)

    tot = []
    pref = []
    for e in range(E):
        vals = plsc.load_gather(allhist_v, [iota * L + e])
        tot.append(jnp.sum(vals))
        pref.append(jnp.sum(jnp.where(iota < wid, vals, 0)))
    cur = []
    acc_pad = jnp.int32(0)
    for e in range(E):
        cur.append(acc_pad + pref[e])
        acc_pad = acc_pad + ((tot[e] + TM - 1) // TM) * TM

    for j in range(TT // L):
        c16 = code_v[pl.ds(j * L, L)]
        e0 = c16 // E
        slot = jnp.zeros((L,), jnp.int32)
        for e in range(E):
            m = e0 == e
            mi = m.astype(jnp.int32)
            excl = plsc.cumsum(mi) - mi
            slot = jnp.where(m, cur[e] + excl, slot)
            cur[e] = cur[e] + jnp.sum(mi)
        s0_v[j * L // CH, pl.ds(j * L % CH, L)] = slot
    for j in range(TT // L):
        c16 = code_v[pl.ds(j * L, L)]
        e0 = c16 // E
        e1 = c16 - e0 * E
        slot = jnp.zeros((L,), jnp.int32)
        for e in range(E):
            m = e1 == e
            mi = m.astype(jnp.int32)
            excl = plsc.cumsum(mi) - mi
            slot = jnp.where(m, cur[e] + excl, slot)
            cur[e] = cur[e] + jnp.sum(mi)
        s1_v[j * L // CH, pl.ds(j * L % CH, L)] = slot

    for c in range(TT // CH):
        pltpu.sync_copy(s0_v.at[c], s0_hbm.at[2 * wid + c])
        pltpu.sync_copy(s1_v.at[c], s1_hbm.at[2 * wid + c])

    for c in range(TT // CH):
        pltpu.sync_copy(x_hbm.at[pl.ds(base_tok + c * CH, CH)], rows_v)
        d0 = pltpu.async_copy(rows_v, xs_hbm.at[s0_v.at[c]], sem0)
        d1 = pltpu.async_copy(rows_v, xs_hbm.at[s1_v.at[c]], sem1)
        d0.wait()
        d1.wait()

    @pl.when(wid == 0)
    def _():
        j0 = iota
        j1 = iota + L
        te0 = jnp.zeros((L,), jnp.int32)
        te1 = jnp.zeros((L,), jnp.int32)
        pt = jnp.int32(0)
        for e in range(E):
            pt_eff = jnp.where(tot[e] > 0, pt, jnp.int32(2 * L))
            te0 = jnp.where(j0 >= pt_eff, e, te0)
            te1 = jnp.where(j1 >= pt_eff, e, te1)
            pt = pt + (tot[e] + TM - 1) // TM
        te_v[pl.ds(0, L)] = te0
        te_v[pl.ds(L, L)] = te1
        tv_v[pl.ds(0, L)] = (j0 < pt).astype(jnp.int32)
        tv_v[pl.ds(L, L)] = (j1 < pt).astype(jnp.int32)
        pltpu.sync_copy(te_v, te_hbm)
        pltpu.sync_copy(tv_v, tv_hbm)


def _dispatch(code_flat, x):
    mesh = plsc.VectorSubcoreMesh(core_axis_name="c", subcore_axis_name="s",
                                  num_cores=1)
    out_type = [
        jax.ShapeDtypeStruct((NTOT, D), jnp.float32),
        jax.ShapeDtypeStruct((2 * NSC, CH), jnp.int32),
        jax.ShapeDtypeStruct((2 * NSC, CH), jnp.int32),
        jax.ShapeDtypeStruct((2 * L,), jnp.int32),
        jax.ShapeDtypeStruct((2 * L,), jnp.int32),
    ]
    scratch = [
        pltpu.VMEM((TT,), jnp.int32),
        pltpu.VMEM((L,), jnp.int32),
        pltpu.VMEM((NSC * L,), jnp.int32),
        pltpu.VMEM((TT // CH, CH), jnp.int32),
        pltpu.VMEM((TT // CH, CH), jnp.int32),
        pltpu.VMEM((CH, D), jnp.float32),
        pltpu.VMEM((2 * L,), jnp.int32),
        pltpu.VMEM((2 * L,), jnp.int32),
        pltpu.VMEM_SHARED((NSC * L,), jnp.int32),
        pltpu.SemaphoreType.DMA,
        pltpu.SemaphoreType.DMA,
    ]
    f = functools.partial(pl.kernel, mesh=mesh, out_type=out_type,
                          scratch_types=scratch,
                          compiler_params=pltpu.CompilerParams(needs_layout_passes=False))(_dispatch_body)
    return f(code_flat, x)


def _gmm_body(te_ref, tv_ref, x_ref, wga_ref, wgb_ref, wua_ref, wub_ref,
              wda_ref, wdb_ref, y_ref):
    i = pl.program_id(0)

    @pl.when(tv_ref[i] != 0)
    def _():
        xb = x_ref[...]
        ga = jnp.dot(xb, wga_ref[0], preferred_element_type=jnp.float32)
        ua = jnp.dot(xb, wua_ref[0], preferred_element_type=jnp.float32)
        ha = (ga * jax.nn.sigmoid(ga)) * ua
        ya = jnp.dot(ha, wda_ref[0], preferred_element_type=jnp.float32)
        gb = jnp.dot(xb, wgb_ref[0], preferred_element_type=jnp.float32)
        ub = jnp.dot(xb, wub_ref[0], preferred_element_type=jnp.float32)
        hb = (gb * jax.nn.sigmoid(gb)) * ub
        y_ref[...] = ya + jnp.dot(hb, wdb_ref[0], preferred_element_type=jnp.float32)


def _gmm(tile_expert, tile_valid, x_sorted, wg, wu, wd):
    grid_spec = pltpu.PrefetchScalarGridSpec(
        num_scalar_prefetch=2,
        grid=(NT,),
        in_specs=[
            pl.BlockSpec((TM, D), lambda i, te, tv: (i, 0)),
            pl.BlockSpec((1, D, F // 2), lambda i, te, tv: (te[i], 0, 0)),
            pl.BlockSpec((1, D, F // 2), lambda i, te, tv: (te[i], 0, 1)),
            pl.BlockSpec((1, D, F // 2), lambda i, te, tv: (te[i], 0, 0)),
            pl.BlockSpec((1, D, F // 2), lambda i, te, tv: (te[i], 0, 1)),
            pl.BlockSpec((1, F // 2, D), lambda i, te, tv: (te[i], 0, 0)),
            pl.BlockSpec((1, F // 2, D), lambda i, te, tv: (te[i], 1, 0)),
        ],
        out_specs=pl.BlockSpec((TM, D), lambda i, te, tv: (i, 0)),
    )
    return pl.pallas_call(
        _gmm_body,
        grid_spec=grid_spec,
        out_shape=jax.ShapeDtypeStruct((NTOT, D), jnp.float32),
        compiler_params=pltpu.CompilerParams(vmem_limit_bytes=100 * 1024 * 1024),
    )(tile_expert, tile_valid, x_sorted, wg, wg, wu, wu, wd, wd)


NW4 = 32
TT4 = T // NW4
CH4 = 32


def _combine_body(y_hbm, s0_hbm, s1_hbm, w1_hbm,
                  out_hbm,
                  i0_v, i1_v, w_v, buf0, buf1, out_v, sem0, sem1):
    cid = lax.axis_index("c")
    sid = lax.axis_index("s")
    w4 = sid * 2 + cid
    base_tok = w4 * TT4

    pltpu.sync_copy(s0_hbm.at[w4], i0_v)
    pltpu.sync_copy(s1_hbm.at[w4], i1_v)
    pltpu.sync_copy(w1_hbm.at[pl.ds(base_tok, TT4)], w_v)

    for c in range(TT4 // CH4):
        d0 = pltpu.async_copy(y_hbm.at[i0_v.at[c]], buf0, sem0)
        d1 = pltpu.async_copy(y_hbm.at[i1_v.at[c]], buf1, sem1)
        d0.wait()
        d1.wait()

        @pl.loop(0, CH4)
        def _(r):
            ridx = jnp.full((L,), c * CH4 + r, jnp.int32)
            w0 = plsc.load_gather(w_v, [ridx])
            w1m = 1.0 - w0
            for j in range(D // L):
                sl = pl.ds(j * L, L)
                out_v[r, sl] = buf0[r, sl] * w0 + buf1[r, sl] * w1m

        pltpu.sync_copy(out_v, out_hbm.at[pl.ds(base_tok + c * CH4, CH4)])


def _combine(y, s0, s1, w1_flat):
    mesh = plsc.VectorSubcoreMesh(core_axis_name="c", subcore_axis_name="s",
                                  num_cores=2)
    out_type = jax.ShapeDtypeStruct((T, D), jnp.float32)
    scratch = [
        pltpu.VMEM((TT4 // CH4, CH4), jnp.int32),
        pltpu.VMEM((TT4 // CH4, CH4), jnp.int32),
        pltpu.VMEM((TT4,), jnp.float32),
        pltpu.VMEM((CH4, D), jnp.float32),
        pltpu.VMEM((CH4, D), jnp.float32),
        pltpu.VMEM((CH4, D), jnp.float32),
        pltpu.SemaphoreType.DMA,
        pltpu.SemaphoreType.DMA,
    ]
    f = functools.partial(pl.kernel, mesh=mesh, out_type=out_type,
                          scratch_types=scratch,
                          compiler_params=pltpu.CompilerParams(needs_layout_passes=False))(_combine_body)
    return f(y, s0.reshape(NW4, TT4 // CH4, CH4), s1.reshape(NW4, TT4 // CH4, CH4), w1_flat)


def kernel(x_TD, W_router, kernel_gating_EDF, kernel_up_proj_EDF, kernel_down_proj_EFD):
    x = jnp.asarray(x_TD, jnp.float32)
    wr_pad = jnp.zeros((D, 128), jnp.float32).at[:, :E].set(W_router)
    wg = kernel_gating_EDF
    wu = kernel_up_proj_EDF
    wd = kernel_down_proj_EFD

    code, w1 = _router(x, wr_pad)
    x_sorted, s0, s1, tile_expert, tile_valid = _dispatch(code.reshape(T), x)
    y = _gmm(tile_expert, tile_valid, x_sorted, wg, wu, wd)
    return _combine(y, s0, s1, w1.reshape(T))


# speedup vs baseline: 1.0038x; 1.0038x over previous
import functools

import jax
import jax.numpy as jnp
from jax import lax
from jax.experimental import pallas as pl
from jax.experimental.pallas import tpu as pltpu
from jax.experimental.pallas import tpu_sc as plsc

T, D, E, F, TOP_K = 2048, 1024, 8, 2048, 2
TM = 512
NT = T * TOP_K // TM + E
NTOT = NT * TM
NSC = 16
TT = T // NSC
CH = 64
L = 16


def _router_body(x_ref, wr_ref, code_ref, w1_ref):
    x = x_ref[...]
    logits = jnp.dot(x, wr_ref[...], preferred_element_type=jnp.float32)
    lane = lax.broadcasted_iota(jnp.int32, logits.shape, 1)
    logits = jnp.where(lane < E, logits, -jnp.inf)
    m1 = jnp.max(logits, axis=1, keepdims=True)
    i1 = jnp.min(jnp.where(logits == m1, lane, E), axis=1, keepdims=True)
    logits2 = jnp.where(lane == i1, -jnp.inf, logits)
    m2 = jnp.max(logits2, axis=1, keepdims=True)
    i2 = jnp.min(jnp.where(logits2 == m2, lane, E), axis=1, keepdims=True)
    w1 = 1.0 / (1.0 + jnp.exp(m2 - m1))
    code_ref[...] = i1 * E + i2
    w1_ref[...] = w1


def _router(x_TD, W_router_pad):
    return pl.pallas_call(
        _router_body,
        grid=(T // TM,),
        in_specs=[
            pl.BlockSpec((TM, D), lambda i: (i, 0)),
            pl.BlockSpec((D, 128), lambda i: (0, 0)),
        ],
        out_specs=[
            pl.BlockSpec((TM, 1), lambda i: (i, 0)),
            pl.BlockSpec((TM, 1), lambda i: (i, 0)),
        ],
        out_shape=[
            jax.ShapeDtypeStruct((T, 1), jnp.int32),
            jax.ShapeDtypeStruct((T, 1), jnp.float32),
        ],
    )(x_TD, W_router_pad)


ND = 32
TTD = T // ND


def _dispatch_body(code_hbm, x_hbm,
                   xs_hbm, s0_hbm, s1_hbm, te_hbm, tv_hbm,
                   code_v, s0_v, s1_v, si0_v, si1_v, rows_v,
                   te_v, tv_v, sem0, sem1):
    cid = lax.axis_index("c")
    sid = lax.axis_index("s")
    wid = sid * 2 + cid
    base_tok = wid * TTD
    iota = lax.iota(jnp.int32, L)

    pltpu.sync_copy(code_hbm, code_v)
    tot_acc = [jnp.zeros((L,), jnp.int32) for _ in range(E)]
    pref_acc = [jnp.zeros((L,), jnp.int32) for _ in range(E)]
    for j in range(T // L):
        c16 = code_v[pl.ds(j * L, L)]
        e0 = c16 // E
        e1 = c16 - e0 * E
        inpref = (j * L + iota) < base_tok
        for e in range(E):
            m = (e0 == e).astype(jnp.int32) + (e1 == e).astype(jnp.int32)
            tot_acc[e] = tot_acc[e] + m
            pref_acc[e] = pref_acc[e] + jnp.where(inpref, m, 0)
    tot = [jnp.sum(tot_acc[e]) for e in range(E)]
    cur = []
    acc_pad = jnp.int32(0)
    for e in range(E):
        cur.append(acc_pad + jnp.sum(pref_acc[e]))
        acc_pad = acc_pad + ((tot[e] + TM - 1) // TM) * TM

    for j in range(TTD // L):
        c16 = code_v[pl.ds(base_tok + j * L, L)]
        e0 = c16 // E
        slot = jnp.zeros((L,), jnp.int32)
        for e in range(E):
            m = e0 == e
            mi = m.astype(jnp.int32)
            excl = plsc.cumsum(mi) - mi
            slot = jnp.where(m, cur[e] + excl, slot)
            cur[e] = cur[e] + jnp.sum(mi)
        s0_v[j // 2, pl.ds((j % 2) * L, L)] = slot
        si0_v[0, pl.ds(j * L, L)] = slot
    for j in range(TTD // L):
        c16 = code_v[pl.ds(base_tok + j * L, L)]
        e0 = c16 // E
        e1 = c16 - e0 * E
        slot = jnp.zeros((L,), jnp.int32)
        for e in range(E):
            m = e1 == e
            mi = m.astype(jnp.int32)
            excl = plsc.cumsum(mi) - mi
            slot = jnp.where(m, cur[e] + excl, slot)
            cur[e] = cur[e] + jnp.sum(mi)
        s1_v[j // 2, pl.ds((j % 2) * L, L)] = slot
        si1_v[0, pl.ds(j * L, L)] = slot

    pltpu.sync_copy(s0_v, s0_hbm.at[wid])
    pltpu.sync_copy(s1_v, s1_hbm.at[wid])

    pltpu.sync_copy(x_hbm.at[pl.ds(base_tok, TTD)], rows_v)
    d0 = pltpu.async_copy(rows_v, xs_hbm.at[si0_v.at[0]], sem0)
    d1 = pltpu.async_copy(rows_v, xs_hbm.at[si1_v.at[0]], sem1)
    d0.wait()
    d1.wait()

    @pl.when(wid == 0)
    def _():
        j0 = iota
        j1 = iota + L
        te0 = jnp.zeros((L,), jnp.int32)
        te1 = jnp.zeros((L,), jnp.int32)
        pt = jnp.int32(0)
        for e in range(E):
            pt_eff = jnp.where(tot[e] > 0, pt, jnp.int32(2 * L))
            te0 = jnp.where(j0 >= pt_eff, e, te0)
            te1 = jnp.where(j1 >= pt_eff, e, te1)
            pt = pt + (tot[e] + TM - 1) // TM
        te_v[pl.ds(0, L)] = te0
        te_v[pl.ds(L, L)] = te1
        tv_v[pl.ds(0, L)] = (j0 < pt).astype(jnp.int32)
        tv_v[pl.ds(L, L)] = (j1 < pt).astype(jnp.int32)
        pltpu.sync_copy(te_v, te_hbm)
        pltpu.sync_copy(tv_v, tv_hbm)


def _dispatch(code_flat, x):
    mesh = plsc.VectorSubcoreMesh(core_axis_name="c", subcore_axis_name="s",
                                  num_cores=2)
    out_type = [
        jax.ShapeDtypeStruct((NTOT, D), jnp.float32),
        jax.ShapeDtypeStruct((ND, 2, TTD // 2), jnp.int32),
        jax.ShapeDtypeStruct((ND, 2, TTD // 2), jnp.int32),
        jax.ShapeDtypeStruct((2 * L,), jnp.int32),
        jax.ShapeDtypeStruct((2 * L,), jnp.int32),
    ]
    scratch = [
        pltpu.VMEM((T,), jnp.int32),
        pltpu.VMEM((2, TTD // 2), jnp.int32),
        pltpu.VMEM((2, TTD // 2), jnp.int32),
        pltpu.VMEM((1, TTD), jnp.int32),
        pltpu.VMEM((1, TTD), jnp.int32),
        pltpu.VMEM((TTD, D), jnp.float32),
        pltpu.VMEM((2 * L,), jnp.int32),
        pltpu.VMEM((2 * L,), jnp.int32),
        pltpu.SemaphoreType.DMA,
        pltpu.SemaphoreType.DMA,
    ]
    f = functools.partial(pl.kernel, mesh=mesh, out_type=out_type,
                          scratch_types=scratch,
                          compiler_params=pltpu.CompilerParams(needs_layout_passes=False))(_dispatch_body)
    return f(code_flat, x)


def _gmm_body(te_ref, tv_ref, x_ref, wga_ref, wgb_ref, wua_ref, wub_ref,
              wda_ref, wdb_ref, y_ref):
    i = pl.program_id(0)

    @pl.when(tv_ref[i] != 0)
    def _():
        xb = x_ref[...]
        ga = jnp.dot(xb, wga_ref[0], preferred_element_type=jnp.float32)
        ua = jnp.dot(xb, wua_ref[0], preferred_element_type=jnp.float32)
        ha = (ga * jax.nn.sigmoid(ga)) * ua
        ya = jnp.dot(ha, wda_ref[0], preferred_element_type=jnp.float32)
        gb = jnp.dot(xb, wgb_ref[0], preferred_element_type=jnp.float32)
        ub = jnp.dot(xb, wub_ref[0], preferred_element_type=jnp.float32)
        hb = (gb * jax.nn.sigmoid(gb)) * ub
        y_ref[...] = ya + jnp.dot(hb, wdb_ref[0], preferred_element_type=jnp.float32)


def _gmm(tile_expert, tile_valid, x_sorted, wg, wu, wd):
    grid_spec = pltpu.PrefetchScalarGridSpec(
        num_scalar_prefetch=2,
        grid=(NT,),
        in_specs=[
            pl.BlockSpec((TM, D), lambda i, te, tv: (i, 0)),
            pl.BlockSpec((1, D, F // 2), lambda i, te, tv: (te[i], 0, 0)),
            pl.BlockSpec((1, D, F // 2), lambda i, te, tv: (te[i], 0, 1)),
            pl.BlockSpec((1, D, F // 2), lambda i, te, tv: (te[i], 0, 0)),
            pl.BlockSpec((1, D, F // 2), lambda i, te, tv: (te[i], 0, 1)),
            pl.BlockSpec((1, F // 2, D), lambda i, te, tv: (te[i], 0, 0)),
            pl.BlockSpec((1, F // 2, D), lambda i, te, tv: (te[i], 1, 0)),
        ],
        out_specs=pl.BlockSpec((TM, D), lambda i, te, tv: (i, 0)),
    )
    return pl.pallas_call(
        _gmm_body,
        grid_spec=grid_spec,
        out_shape=jax.ShapeDtypeStruct((NTOT, D), jnp.float32),
        compiler_params=pltpu.CompilerParams(vmem_limit_bytes=100 * 1024 * 1024),
    )(tile_expert, tile_valid, x_sorted, wg, wg, wu, wu, wd, wd)


NW4 = 32
TT4 = T // NW4
CH4 = 32


def _combine_body(y_hbm, s0_hbm, s1_hbm, w1_hbm,
                  out_hbm,
                  i0_v, i1_v, w_v, buf0, buf1, out_v, sem0, sem1):
    cid = lax.axis_index("c")
    sid = lax.axis_index("s")
    w4 = sid * 2 + cid
    base_tok = w4 * TT4

    pltpu.sync_copy(s0_hbm.at[w4], i0_v)
    pltpu.sync_copy(s1_hbm.at[w4], i1_v)
    pltpu.sync_copy(w1_hbm.at[pl.ds(base_tok, TT4)], w_v)

    for c in range(TT4 // CH4):
        d0 = pltpu.async_copy(y_hbm.at[i0_v.at[c]], buf0, sem0)
        d1 = pltpu.async_copy(y_hbm.at[i1_v.at[c]], buf1, sem1)
        d0.wait()
        d1.wait()

        @pl.loop(0, CH4)
        def _(r):
            ridx = jnp.full((L,), c * CH4 + r, jnp.int32)
            w0 = plsc.load_gather(w_v, [ridx])
            w1m = 1.0 - w0
            for j in range(D // L):
                sl = pl.ds(j * L, L)
                out_v[r, sl] = buf0[r, sl] * w0 + buf1[r, sl] * w1m

        pltpu.sync_copy(out_v, out_hbm.at[pl.ds(base_tok + c * CH4, CH4)])


def _combine(y, s0, s1, w1_flat):
    mesh = plsc.VectorSubcoreMesh(core_axis_name="c", subcore_axis_name="s",
                                  num_cores=2)
    out_type = jax.ShapeDtypeStruct((T, D), jnp.float32)
    scratch = [
        pltpu.VMEM((TT4 // CH4, CH4), jnp.int32),
        pltpu.VMEM((TT4 // CH4, CH4), jnp.int32),
        pltpu.VMEM((TT4,), jnp.float32),
        pltpu.VMEM((CH4, D), jnp.float32),
        pltpu.VMEM((CH4, D), jnp.float32),
        pltpu.VMEM((CH4, D), jnp.float32),
        pltpu.SemaphoreType.DMA,
        pltpu.SemaphoreType.DMA,
    ]
    f = functools.partial(pl.kernel, mesh=mesh, out_type=out_type,
                          scratch_types=scratch,
                          compiler_params=pltpu.CompilerParams(needs_layout_passes=False))(_combine_body)
    return f(y, s0, s1, w1_flat)


def kernel(x_TD, W_router, kernel_gating_EDF, kernel_up_proj_EDF, kernel_down_proj_EFD):
    x = jnp.asarray(x_TD, jnp.float32)
    wr_pad = jnp.zeros((D, 128), jnp.float32).at[:, :E].set(W_router)
    wg = kernel_gating_EDF
    wu = kernel_up_proj_EDF
    wd = kernel_down_proj_EFD

    code, w1 = _router(x, wr_pad)
    x_sorted, s0, s1, tile_expert, tile_valid = _dispatch(code.reshape(T), x)
    y = _gmm(tile_expert, tile_valid, x_sorted, wg, wu, wd)
    return _combine(y, s0, s1, w1.reshape(T))


# speedup vs baseline: 1.0217x; 1.0178x over previous
import functools

import jax
import jax.numpy as jnp
from jax import lax
from jax.experimental import pallas as pl
from jax.experimental.pallas import tpu as pltpu
from jax.experimental.pallas import tpu_sc as plsc

T, D, E, F, TOP_K = 2048, 1024, 8, 2048, 2
TM = 512
NT = T * TOP_K // TM + E
NTOT = NT * TM
NSC = 16
TT = T // NSC
CH = 64
L = 16


def _router_body(x_ref, wr_ref, code_ref, w1_ref):
    x = x_ref[...]
    logits = jnp.dot(x, wr_ref[...], preferred_element_type=jnp.float32)
    lane = lax.broadcasted_iota(jnp.int32, logits.shape, 1)
    logits = jnp.where(lane < E, logits, -jnp.inf)
    m1 = jnp.max(logits, axis=1, keepdims=True)
    i1 = jnp.min(jnp.where(logits == m1, lane, E), axis=1, keepdims=True)
    logits2 = jnp.where(lane == i1, -jnp.inf, logits)
    m2 = jnp.max(logits2, axis=1, keepdims=True)
    i2 = jnp.min(jnp.where(logits2 == m2, lane, E), axis=1, keepdims=True)
    w1 = 1.0 / (1.0 + jnp.exp(m2 - m1))
    code_ref[...] = i1 * E + i2
    w1_ref[...] = w1


def _router(x_TD, W_router_pad):
    return pl.pallas_call(
        _router_body,
        grid=(T // TM,),
        in_specs=[
            pl.BlockSpec((TM, D), lambda i: (i, 0)),
            pl.BlockSpec((D, 128), lambda i: (0, 0)),
        ],
        out_specs=[
            pl.BlockSpec((TM, 1), lambda i: (i, 0)),
            pl.BlockSpec((TM, 1), lambda i: (i, 0)),
        ],
        out_shape=[
            jax.ShapeDtypeStruct((T, 1), jnp.int32),
            jax.ShapeDtypeStruct((T, 1), jnp.float32),
        ],
    )(x_TD, W_router_pad)


ND = 32
TTD = T // ND


def _dispatch_body(code_hbm, x_hbm,
                   xs_hbm, s0_hbm, s1_hbm, te_hbm, tv_hbm,
                   code_v, s0_v, s1_v, si0_v, si1_v, rows_v,
                   te_v, tv_v, sem0, sem1):
    cid = lax.axis_index("c")
    sid = lax.axis_index("s")
    wid = sid * 2 + cid
    base_tok = wid * TTD
    iota = lax.iota(jnp.int32, L)

    pltpu.sync_copy(code_hbm, code_v)
    tot_acc = [jnp.zeros((L,), jnp.int32) for _ in range(E)]
    pref_acc = [jnp.zeros((L,), jnp.int32) for _ in range(E)]
    for j in range(T // L):
        c16 = code_v[pl.ds(j * L, L)]
        e0 = c16 // E
        e1 = c16 - e0 * E
        inpref = (j * L + iota) < base_tok
        for e in range(E):
            m = (e0 == e).astype(jnp.int32) + (e1 == e).astype(jnp.int32)
            tot_acc[e] = tot_acc[e] + m
            pref_acc[e] = pref_acc[e] + jnp.where(inpref, m, 0)
    tot = [jnp.sum(tot_acc[e]) for e in range(E)]
    cur = []
    acc_pad = jnp.int32(0)
    for e in range(E):
        cur.append(acc_pad + jnp.sum(pref_acc[e]))
        acc_pad = acc_pad + ((tot[e] + TM - 1) // TM) * TM

    for j in range(TTD // L):
        c16 = code_v[pl.ds(base_tok + j * L, L)]
        e0 = c16 // E
        slot = jnp.zeros((L,), jnp.int32)
        for e in range(E):
            m = e0 == e
            mi = m.astype(jnp.int32)
            excl = plsc.cumsum(mi) - mi
            slot = jnp.where(m, cur[e] + excl, slot)
            cur[e] = cur[e] + jnp.sum(mi)
        s0_v[j // 2, pl.ds((j % 2) * L, L)] = slot
        si0_v[0, pl.ds(j * L, L)] = slot
    for j in range(TTD // L):
        c16 = code_v[pl.ds(base_tok + j * L, L)]
        e0 = c16 // E
        e1 = c16 - e0 * E
        slot = jnp.zeros((L,), jnp.int32)
        for e in range(E):
            m = e1 == e
            mi = m.astype(jnp.int32)
            excl = plsc.cumsum(mi) - mi
            slot = jnp.where(m, cur[e] + excl, slot)
            cur[e] = cur[e] + jnp.sum(mi)
        s1_v[j // 2, pl.ds((j % 2) * L, L)] = slot
        si1_v[0, pl.ds(j * L, L)] = slot

    pltpu.sync_copy(s0_v, s0_hbm.at[wid])
    pltpu.sync_copy(s1_v, s1_hbm.at[wid])

    pltpu.sync_copy(x_hbm.at[pl.ds(base_tok, TTD)], rows_v)
    d0 = pltpu.async_copy(rows_v, xs_hbm.at[si0_v.at[0]], sem0)
    d1 = pltpu.async_copy(rows_v, xs_hbm.at[si1_v.at[0]], sem1)
    d0.wait()
    d1.wait()

    @pl.when(wid == 0)
    def _():
        j0 = iota
        j1 = iota + L
        te0 = jnp.zeros((L,), jnp.int32)
        te1 = jnp.zeros((L,), jnp.int32)
        pt = jnp.int32(0)
        for e in range(E):
            pt_eff = jnp.where(tot[e] > 0, pt, jnp.int32(2 * L))
            te0 = jnp.where(j0 >= pt_eff, e, te0)
            te1 = jnp.where(j1 >= pt_eff, e, te1)
            pt = pt + (tot[e] + TM - 1) // TM
        te_v[pl.ds(0, L)] = te0
        te_v[pl.ds(L, L)] = te1
        tv_v[pl.ds(0, L)] = (j0 < pt).astype(jnp.int32)
        tv_v[pl.ds(L, L)] = (j1 < pt).astype(jnp.int32)
        pltpu.sync_copy(te_v, te_hbm)
        pltpu.sync_copy(tv_v, tv_hbm)


def _dispatch(code_flat, x):
    mesh = plsc.VectorSubcoreMesh(core_axis_name="c", subcore_axis_name="s",
                                  num_cores=2)
    out_type = [
        jax.ShapeDtypeStruct((NTOT, D), jnp.float32),
        jax.ShapeDtypeStruct((ND, 2, TTD // 2), jnp.int32),
        jax.ShapeDtypeStruct((ND, 2, TTD // 2), jnp.int32),
        jax.ShapeDtypeStruct((2 * L,), jnp.int32),
        jax.ShapeDtypeStruct((2 * L,), jnp.int32),
    ]
    scratch = [
        pltpu.VMEM((T,), jnp.int32),
        pltpu.VMEM((2, TTD // 2), jnp.int32),
        pltpu.VMEM((2, TTD // 2), jnp.int32),
        pltpu.VMEM((1, TTD), jnp.int32),
        pltpu.VMEM((1, TTD), jnp.int32),
        pltpu.VMEM((TTD, D), jnp.float32),
        pltpu.VMEM((2 * L,), jnp.int32),
        pltpu.VMEM((2 * L,), jnp.int32),
        pltpu.SemaphoreType.DMA,
        pltpu.SemaphoreType.DMA,
    ]
    f = functools.partial(pl.kernel, mesh=mesh, out_type=out_type,
                          scratch_types=scratch,
                          compiler_params=pltpu.CompilerParams(needs_layout_passes=False))(_dispatch_body)
    return f(code_flat, x)


def _gmm_body(te_ref, tv_ref, x_ref, wga_ref, wgb_ref, wua_ref, wub_ref,
              wda_ref, wdb_ref, y_ref):
    i = pl.program_id(0)

    @pl.when(tv_ref[i] != 0)
    def _():
        xb = x_ref[...]
        ga = jnp.dot(xb, wga_ref[0], preferred_element_type=jnp.float32)
        ua = jnp.dot(xb, wua_ref[0], preferred_element_type=jnp.float32)
        ha = (ga * jax.nn.sigmoid(ga)) * ua
        ya = jnp.dot(ha, wda_ref[0], preferred_element_type=jnp.float32)
        gb = jnp.dot(xb, wgb_ref[0], preferred_element_type=jnp.float32)
        ub = jnp.dot(xb, wub_ref[0], preferred_element_type=jnp.float32)
        hb = (gb * jax.nn.sigmoid(gb)) * ub
        y_ref[...] = ya + jnp.dot(hb, wdb_ref[0], preferred_element_type=jnp.float32)


def _gmm(tile_expert, tile_valid, x_sorted, wg, wu, wd):
    grid_spec = pltpu.PrefetchScalarGridSpec(
        num_scalar_prefetch=2,
        grid=(NT,),
        in_specs=[
            pl.BlockSpec((TM, D), lambda i, te, tv: (i, 0)),
            pl.BlockSpec((1, D, F // 2), lambda i, te, tv: (te[i], 0, 0)),
            pl.BlockSpec((1, D, F // 2), lambda i, te, tv: (te[i], 0, 1)),
            pl.BlockSpec((1, D, F // 2), lambda i, te, tv: (te[i], 0, 0)),
            pl.BlockSpec((1, D, F // 2), lambda i, te, tv: (te[i], 0, 1)),
            pl.BlockSpec((1, F // 2, D), lambda i, te, tv: (te[i], 0, 0)),
            pl.BlockSpec((1, F // 2, D), lambda i, te, tv: (te[i], 1, 0)),
        ],
        out_specs=pl.BlockSpec((TM, D), lambda i, te, tv: (i, 0)),
    )
    return pl.pallas_call(
        _gmm_body,
        grid_spec=grid_spec,
        out_shape=jax.ShapeDtypeStruct((NTOT, D), jnp.float32),
        compiler_params=pltpu.CompilerParams(vmem_limit_bytes=100 * 1024 * 1024),
    )(tile_expert, tile_valid, x_sorted, wg, wg, wu, wu, wd, wd)


NW4 = 32
TT4 = T // NW4
CH4 = 16


def _combine_body(y_hbm, s0_hbm, s1_hbm, w1_hbm,
                  out_hbm,
                  i0_v, i1_v, w_v, b0a, b1a, b0b, b1b, out_v, sem0, sem1,
                  sem2, sem3):
    cid = lax.axis_index("c")
    sid = lax.axis_index("s")
    w4 = sid * 2 + cid
    base_tok = w4 * TT4
    nch = TT4 // CH4

    pltpu.sync_copy(s0_hbm.at[w4], i0_v)
    pltpu.sync_copy(s1_hbm.at[w4], i1_v)
    pltpu.sync_copy(w1_hbm.at[pl.ds(base_tok, TT4)], w_v)

    bufs0 = [b0a, b0b]
    bufs1 = [b1a, b1b]
    sems0 = [sem0, sem2]
    sems1 = [sem1, sem3]
    dmas = {}
    for c in range(nch + 1):
        if c < nch:
            pr = c % 2
            dmas[c] = (
                pltpu.async_copy(y_hbm.at[i0_v.at[c // 2, pl.ds((c % 2) * CH4, CH4)]],
                                 bufs0[pr], sems0[pr]),
                pltpu.async_copy(y_hbm.at[i1_v.at[c // 2, pl.ds((c % 2) * CH4, CH4)]],
                                 bufs1[pr], sems1[pr]),
            )
        if c >= 1:
            cc = c - 1
            pr = cc % 2
            d0, d1 = dmas.pop(cc)
            d0.wait()
            d1.wait()
            buf0 = bufs0[pr]
            buf1 = bufs1[pr]

            @pl.loop(0, CH4)
            def _(r):
                ridx = jnp.full((L,), cc * CH4 + r, jnp.int32)
                w0 = plsc.load_gather(w_v, [ridx])
                w1m = 1.0 - w0
                for j in range(D // L):
                    sl = pl.ds(j * L, L)
                    out_v[r, sl] = buf0[r, sl] * w0 + buf1[r, sl] * w1m

            pltpu.sync_copy(out_v, out_hbm.at[pl.ds(base_tok + cc * CH4, CH4)])


def _combine(y, s0, s1, w1_flat):
    mesh = plsc.VectorSubcoreMesh(core_axis_name="c", subcore_axis_name="s",
                                  num_cores=2)
    out_type = jax.ShapeDtypeStruct((T, D), jnp.float32)
    scratch = [
        pltpu.VMEM((TT4 // CH4 // 2, 2 * CH4), jnp.int32),
        pltpu.VMEM((TT4 // CH4 // 2, 2 * CH4), jnp.int32),
        pltpu.VMEM((TT4,), jnp.float32),
        pltpu.VMEM((CH4, D), jnp.float32),
        pltpu.VMEM((CH4, D), jnp.float32),
        pltpu.VMEM((CH4, D), jnp.float32),
        pltpu.VMEM((CH4, D), jnp.float32),
        pltpu.VMEM((CH4, D), jnp.float32),
        pltpu.SemaphoreType.DMA,
        pltpu.SemaphoreType.DMA,
        pltpu.SemaphoreType.DMA,
        pltpu.SemaphoreType.DMA,
    ]
    f = functools.partial(pl.kernel, mesh=mesh, out_type=out_type,
                          scratch_types=scratch,
                          compiler_params=pltpu.CompilerParams(needs_layout_passes=False))(_combine_body)
    return f(y, s0, s1, w1_flat)


def kernel(x_TD, W_router, kernel_gating_EDF, kernel_up_proj_EDF, kernel_down_proj_EFD):
    x = jnp.asarray(x_TD, jnp.float32)
    wr_pad = jnp.zeros((D, 128), jnp.float32).at[:, :E].set(W_router)
    wg = kernel_gating_EDF
    wu = kernel_up_proj_EDF
    wd = kernel_down_proj_EFD

    code, w1 = _router(x, wr_pad)
    x_sorted, s0, s1, tile_expert, tile_valid = _dispatch(code.reshape(T), x)
    y = _gmm(tile_expert, tile_valid, x_sorted, wg, wu, wd)
    return _combine(y, s0, s1, w1.reshape(T))
